# baseline (speedup 1.0000x reference)
import functools

import jax
import jax.numpy as jnp
from jax.experimental import pallas as pl
from jax.experimental.pallas import tpu as pltpu

_NUM_EXPERTS = 64
_TOPK = 8
_TEMP = 1.0
_HIDDEN = 4096
_BM = 512


def _router_block(x_ref, w_ref, b_ref, gb_ref, out_ref, ids_ref):
    x = x_ref[...]
    w = w_ref[...]
    gate = jax.lax.dot_general(
        w, x, (((1,), (1,)), ((), ())),
        preferred_element_type=jnp.float32,
    )
    gate = gate * (1.0 / _TEMP) + b_ref[...]
    work = gate + gb_ref[...]
    row = jax.lax.broadcasted_iota(jnp.int32, gate.shape, 0)

    bits = jax.lax.bitcast_convert_type(work, jnp.int32)
    skey = bits ^ ((bits >> 31) & jnp.int32(0x7FFFFFFF))
    key = (skey & jnp.int32(-64)) | (jnp.int32(_NUM_EXPERTS - 1) - row)

    sentinel = jnp.int32(-(2 ** 31))
    ids_rows = []
    for _ in range(_TOPK):
        m = jnp.max(key, axis=0, keepdims=True)
        ids_rows.append(jnp.int32(_NUM_EXPERTS - 1) - (m & jnp.int32(63)))
        key = jnp.where(key == m, sentinel, key)
    ids = jnp.concatenate(ids_rows, axis=0)
    ids_ref[...] = jnp.transpose(ids, (1, 0))

    selected = key == sentinel
    e = jnp.where(selected, jnp.exp(gate), 0.0)
    out = e / jnp.sum(e, axis=0, keepdims=True)
    out_ref[...] = jnp.transpose(out, (1, 0))


@functools.partial(jax.jit, static_argnames=())
def kernel(x, W, b, gate_bias):
    B, S, H = x.shape
    M = B * S
    x2 = x.reshape(M, H)
    b2 = b.reshape(_NUM_EXPERTS, 1)
    gb2 = gate_bias.reshape(_NUM_EXPERTS, 1)

    grid = (M // _BM,)
    out_t, ids_t = pl.pallas_call(
        _router_block,
        grid=grid,
        in_specs=[
            pl.BlockSpec((_BM, H), lambda i: (i, 0)),
            pl.BlockSpec((_NUM_EXPERTS, H), lambda i: (0, 0)),
            pl.BlockSpec((_NUM_EXPERTS, 1), lambda i: (0, 0)),
            pl.BlockSpec((_NUM_EXPERTS, 1), lambda i: (0, 0)),
        ],
        out_specs=[
            pl.BlockSpec((_BM, _NUM_EXPERTS), lambda i: (i, 0)),
            pl.BlockSpec((_BM, _TOPK), lambda i: (i, 0)),
        ],
        out_shape=[
            jax.ShapeDtypeStruct((M, _NUM_EXPERTS), jnp.float32),
            jax.ShapeDtypeStruct((M, _TOPK), jnp.int32),
        ],
        compiler_params=pltpu.CompilerParams(
            dimension_semantics=("arbitrary",),
        ),
    )(x2, W, b2, gb2)
    return out_t.reshape(B, S, _NUM_EXPERTS), ids_t.reshape(B, S, _TOPK)


# speedup vs baseline: 1.1653x; 1.1653x over previous
import functools

import jax
import jax.numpy as jnp
from jax.experimental import pallas as pl
from jax.experimental.pallas import tpu as pltpu

_NUM_EXPERTS = 64
_TOPK = 8
_TEMP = 1.0
_HIDDEN = 4096
_BM = 1024


def _router_block(x_ref, w_ref, b_ref, gb_ref, out_ref, ids_ref):
    x = x_ref[...]
    w = w_ref[...]
    gate = jax.lax.dot_general(
        w, x, (((1,), (1,)), ((), ())),
        preferred_element_type=jnp.float32,
    )
    gate = gate * (1.0 / _TEMP) + b_ref[...]
    work = gate + gb_ref[...]
    row = jax.lax.broadcasted_iota(jnp.int32, gate.shape, 0)

    bits = jax.lax.bitcast_convert_type(work, jnp.int32)
    skey = bits ^ ((bits >> 31) & jnp.int32(0x7FFFFFFF))
    key = (skey & jnp.int32(-64)) | (jnp.int32(_NUM_EXPERTS - 1) - row)

    sentinel = jnp.int32(-(2 ** 31))
    ids_rows = []
    for _ in range(_TOPK):
        m = jnp.max(key, axis=0, keepdims=True)
        ids_rows.append(jnp.int32(_NUM_EXPERTS - 1) - (m & jnp.int32(63)))
        key = jnp.where(key == m, sentinel, key)
    ids_ref[...] = jnp.concatenate(ids_rows, axis=0)

    selected = key == sentinel
    e = jnp.where(selected, jnp.exp(gate), 0.0)
    out_ref[...] = e / jnp.sum(e, axis=0, keepdims=True)


@functools.partial(jax.jit, static_argnames=())
def kernel(x, W, b, gate_bias):
    B, S, H = x.shape
    M = B * S
    x2 = x.reshape(M, H)
    b2 = b.reshape(_NUM_EXPERTS, 1)
    gb2 = gate_bias.reshape(_NUM_EXPERTS, 1)

    grid = (M // _BM,)
    out_t, ids_t = pl.pallas_call(
        _router_block,
        grid=grid,
        in_specs=[
            pl.BlockSpec((_BM, H), lambda i: (i, 0)),
            pl.BlockSpec((_NUM_EXPERTS, H), lambda i: (0, 0)),
            pl.BlockSpec((_NUM_EXPERTS, 1), lambda i: (0, 0)),
            pl.BlockSpec((_NUM_EXPERTS, 1), lambda i: (0, 0)),
        ],
        out_specs=[
            pl.BlockSpec((_NUM_EXPERTS, _BM), lambda i: (0, i)),
            pl.BlockSpec((_TOPK, _BM), lambda i: (0, i)),
        ],
        out_shape=[
            jax.ShapeDtypeStruct((_NUM_EXPERTS, M), jnp.float32),
            jax.ShapeDtypeStruct((_TOPK, M), jnp.int32),
        ],
        compiler_params=pltpu.CompilerParams(
            dimension_semantics=("arbitrary",),
        ),
    )(x2, W, b2, gb2)
    out = out_t.T.reshape(B, S, _NUM_EXPERTS)
    ids = ids_t.T.reshape(B, S, _TOPK)
    return out, ids


# speedup vs baseline: 1.1666x; 1.0011x over previous
import functools

import jax
import jax.numpy as jnp
from jax.experimental import pallas as pl
from jax.experimental.pallas import tpu as pltpu

_NUM_EXPERTS = 64
_TOPK = 8
_TEMP = 1.0
_HIDDEN = 4096
_BM = 1024


def _route(gate, gb):
    work = gate + gb
    row = jax.lax.broadcasted_iota(jnp.int32, gate.shape, 0)

    bits = jax.lax.bitcast_convert_type(work, jnp.int32)
    skey = bits ^ ((bits >> 31) & jnp.int32(0x7FFFFFFF))
    key = (skey & jnp.int32(-64)) | (jnp.int32(_NUM_EXPERTS - 1) - row)

    sentinel = jnp.int32(-(2 ** 31))
    ids_rows = []
    for _ in range(_TOPK):
        m = jnp.max(key, axis=0, keepdims=True)
        ids_rows.append(jnp.int32(_NUM_EXPERTS - 1) - (m & jnp.int32(63)))
        key = jnp.where(key == m, sentinel, key)
    ids = jnp.concatenate(ids_rows, axis=0)

    selected = key == sentinel
    e = jnp.where(selected, jnp.exp(gate), 0.0)
    return e / jnp.sum(e, axis=0, keepdims=True), ids


def _router_block(x_ref, w_ref, b_ref, gb_ref, out_ref, ids_ref):
    w = w_ref[...]
    b = b_ref[...]
    gb = gb_ref[...]
    half = _BM // 2
    gates = []
    for h in range(2):
        xh = x_ref[pl.ds(h * half, half), :]
        g = jax.lax.dot_general(
            w, xh, (((1,), (1,)), ((), ())),
            preferred_element_type=jnp.float32,
        )
        gates.append(g * (1.0 / _TEMP) + b)
    for h in range(2):
        out_h, ids_h = _route(gates[h], gb)
        out_ref[:, pl.ds(h * half, half)] = out_h
        ids_ref[:, pl.ds(h * half, half)] = ids_h


@functools.partial(jax.jit, static_argnames=())
def kernel(x, W, b, gate_bias):
    B, S, H = x.shape
    M = B * S
    x2 = x.reshape(M, H)
    b2 = b.reshape(_NUM_EXPERTS, 1)
    gb2 = gate_bias.reshape(_NUM_EXPERTS, 1)

    grid = (M // _BM,)
    out_t, ids_t = pl.pallas_call(
        _router_block,
        grid=grid,
        in_specs=[
            pl.BlockSpec((_BM, H), lambda i: (i, 0)),
            pl.BlockSpec((_NUM_EXPERTS, H), lambda i: (0, 0)),
            pl.BlockSpec((_NUM_EXPERTS, 1), lambda i: (0, 0)),
            pl.BlockSpec((_NUM_EXPERTS, 1), lambda i: (0, 0)),
        ],
        out_specs=[
            pl.BlockSpec((_NUM_EXPERTS, _BM), lambda i: (0, i)),
            pl.BlockSpec((_TOPK, _BM), lambda i: (0, i)),
        ],
        out_shape=[
            jax.ShapeDtypeStruct((_NUM_EXPERTS, M), jnp.float32),
            jax.ShapeDtypeStruct((_TOPK, M), jnp.int32),
        ],
        compiler_params=pltpu.CompilerParams(
            dimension_semantics=("arbitrary",),
        ),
    )(x2, W, b2, gb2)
    out = out_t.T.reshape(B, S, _NUM_EXPERTS)
    ids = ids_t.T.reshape(B, S, _TOPK)
    return out, ids
